# Initial kernel scaffold; baseline (speedup 1.0000x reference)
import jax
import jax.numpy as jnp
from jax.experimental import pallas as pl


def kernel(input_ids, pos_embedding):
    raise NotImplementedError("write your pallas kernel here")


# speedup vs baseline: 3.6153x; 3.6153x over previous
import functools

import jax
import jax.numpy as jnp
from jax import lax
from jax.experimental import pallas as pl
from jax.experimental.pallas import tpu as pltpu
from jax.experimental.pallas import tpu_sc as plsc

_BATCH = 4
_SEQ = 8192
_DIM = 1024
_NUM_WORKERS = 32
_ROWS_PER_WORKER = _SEQ // _NUM_WORKERS
_CHUNK = 64
_NCHUNKS = _ROWS_PER_WORKER // _CHUNK


def _broadcast_table(pos_embedding):
    mesh = plsc.VectorSubcoreMesh(core_axis_name="c", subcore_axis_name="s")

    @functools.partial(
        pl.kernel,
        mesh=mesh,
        out_type=jax.ShapeDtypeStruct((_BATCH, _SEQ, _DIM), jnp.float32),
        scratch_types=[pltpu.VMEM((_CHUNK, _DIM), jnp.float32)],
    )
    def k(table_hbm, out_hbm, buf):
        wid = lax.axis_index("s") * 2 + lax.axis_index("c")
        base = wid * _ROWS_PER_WORKER
        for i in range(_NCHUNKS):
            row0 = base + i * _CHUNK
            pltpu.sync_copy(table_hbm.at[pl.ds(row0, _CHUNK)], buf)
            for b in range(_BATCH):
                pltpu.sync_copy(buf, out_hbm.at[b, pl.ds(row0, _CHUNK)])

    return k(pos_embedding)


def kernel(input_ids, pos_embedding):
    del input_ids
    return _broadcast_table(pos_embedding)
